# baseline (speedup 1.0000x reference)
import functools

import jax
import jax.numpy as jnp
from jax import lax
from jax.experimental import pallas as pl
from jax.experimental.pallas import tpu as pltpu
from jax.experimental.pallas import tpu_sc as plsc

N = 10000
E = 320000
D = 128
R = 8

NC = 2
NS = 16
NW = NC * NS
LANES = 16

CHUNK = 128
CPT0 = 63
CPT1 = 94
CPT_SUM = CPT0 + CPT1
NCHUNKS = CPT_SUM * NS
E_PAD = NCHUNKS * CHUNK

N_PAD = 10240
ROWS_PER_TILE = N_PAD // NS
ZCHUNK = 32


def _bcast_lane(v, i):
    idx = jnp.full((LANES,), i, jnp.int32)
    return lax.gather(
        v, idx[:, None],
        lax.GatherDimensionNumbers(
            offset_dims=(), collapsed_slice_dims=(0,), start_index_map=(0,)),
        (1,), mode=lax.GatherScatterMode.PROMISE_IN_BOUNDS)


def _z_body(x_ref, w1_ref, w2_ref, b1_ref, z_ref):
    x = x_ref[...]
    b1 = b1_ref[...]
    for r in range(R):
        h = jnp.maximum(
            jnp.dot(x, w1_ref[r], preferred_element_type=jnp.float32) + b1, 0.0)
        z_ref[r] = jnp.dot(h, w2_ref[r], preferred_element_type=jnp.float32)


def _final_body(p_ref, b2_ref, o_ref):
    o_ref[...] = jnp.maximum(p_ref[0] + p_ref[1] + b2_ref[...], 0.0)


def _sc_body(z_hbm, edata_hbm, norm_hbm, out_hbm,
             ebuf, nrmv, gidx, rowsv, zbuf, acc, sem):
    cid = lax.axis_index("c")
    sid = lax.axis_index("s")
    cbase = sid * CPT_SUM + cid * CPT0
    count = jnp.where(cid == 0, CPT0, CPT1)

    def zrow(i, _):
        for q in range(D // LANES):
            zbuf[i, pl.ds(q * LANES, LANES)] = jnp.zeros((LANES,), jnp.float32)
        return _
    lax.fori_loop(0, ZCHUNK, zrow, None)
    arow = sid * ROWS_PER_TILE
    for t in range(ROWS_PER_TILE // ZCHUNK):
        pltpu.sync_copy(zbuf, acc.at[pl.ds(arow + t * ZCHUNK, ZCHUNK)])
    plsc.subcore_barrier()

    def chunk_body(c, _):
        ci = cbase + c
        pltpu.sync_copy(edata_hbm.at[ci], ebuf)
        pltpu.sync_copy(norm_hbm.at[ci], nrmv)
        for j in range(CHUNK // LANES):
            sl = pl.ds(j * LANES, LANES)
            gidx[sl] = ebuf[1, sl] * N + ebuf[0, sl]
        pltpu.async_copy(z_hbm.at[gidx], rowsv, sem).wait()

        def scale16(j, _):
            nv = nrmv[pl.ds(j * LANES, LANES)]
            for i in range(LANES):
                k = j * LANES + i
                nb = _bcast_lane(nv, i)
                for q in range(D // LANES):
                    sl = pl.ds(q * LANES, LANES)
                    rowsv[k, sl] = rowsv[k, sl] * nb
            return _
        lax.fori_loop(0, CHUNK // LANES, scale16, None)
        pltpu.sync_copy(rowsv, acc.at[ebuf.at[2]], add=True)
        return _
    lax.fori_loop(0, count, chunk_body, None)

    plsc.subcore_barrier()
    pltpu.sync_copy(acc.at[pl.ds(arow, ROWS_PER_TILE)],
                    out_hbm.at[cid, pl.ds(arow, ROWS_PER_TILE)])


@jax.jit
def kernel(inputs, edge_index, rel_type, norm, weight1, weight2, bias1, bias2):
    bn = 2000
    z = pl.pallas_call(
        _z_body,
        grid=(N // bn,),
        in_specs=[
            pl.BlockSpec((bn, D), lambda i: (i, 0)),
            pl.BlockSpec((R, D, D), lambda i: (0, 0, 0)),
            pl.BlockSpec((R, D, D), lambda i: (0, 0, 0)),
            pl.BlockSpec((1, D), lambda i: (0, 0)),
        ],
        out_specs=pl.BlockSpec((R, bn, D), lambda i: (0, i, 0)),
        out_shape=jax.ShapeDtypeStruct((R, N, D), jnp.float32),
    )(inputs, weight1, weight2, bias1.reshape(1, D))
    z = z.reshape(R * N, D)

    pad = E_PAD - E
    zi = jnp.zeros((pad,), jnp.int32)
    src = jnp.concatenate([edge_index[0], zi]).reshape(NCHUNKS, CHUNK)
    rel = jnp.concatenate([rel_type, zi]).reshape(NCHUNKS, CHUNK)
    dst = jnp.concatenate([edge_index[1], zi]).reshape(NCHUNKS, CHUNK)
    nrm = jnp.concatenate(
        [norm[:, 0], jnp.zeros((pad,), jnp.float32)]).reshape(NCHUNKS, CHUNK)
    edata = jnp.stack([src, rel, dst], axis=1)

    sc_edges = pl.kernel(
        _sc_body,
        out_type=jax.ShapeDtypeStruct((NC, N_PAD, D), jnp.float32),
        mesh=plsc.VectorSubcoreMesh(core_axis_name="c", subcore_axis_name="s"),
        scratch_types=[
            pltpu.VMEM((3, CHUNK), jnp.int32),
            pltpu.VMEM((CHUNK,), jnp.float32),
            pltpu.VMEM((CHUNK,), jnp.int32),
            pltpu.VMEM((CHUNK, D), jnp.float32),
            pltpu.VMEM((ZCHUNK, D), jnp.float32),
            pltpu.VMEM_SHARED((N_PAD, D), jnp.float32),
            pltpu.SemaphoreType.DMA,
        ],
    )
    partial = sc_edges(z, edata, nrm)

    bm = 2000
    h = pl.pallas_call(
        _final_body,
        grid=(N // bm,),
        in_specs=[
            pl.BlockSpec((NC, bm, D), lambda i: (0, i, 0)),
            pl.BlockSpec((1, D), lambda i: (0, 0)),
        ],
        out_specs=pl.BlockSpec((bm, D), lambda i: (i, 0)),
        out_shape=jax.ShapeDtypeStruct((N, D), jnp.float32),
    )(partial, bias2.reshape(1, D))
    return h
